# Initial kernel scaffold; baseline (speedup 1.0000x reference)
import jax
import jax.numpy as jnp
from jax.experimental import pallas as pl


def kernel(user_embed, item_embed, adj_row, adj_col, adj_val):
    raise NotImplementedError("write your pallas kernel here")


# speedup vs baseline: 2.2283x; 2.2283x over previous
import functools

import jax
import jax.numpy as jnp
from jax import lax
from jax.experimental import pallas as pl
from jax.experimental.pallas import tpu as pltpu
from jax.experimental.pallas import tpu_sc as plsc

N_USERS = 50000
N_NODES = 100000
D = 64
E = 1250000

NC = 2
NS = 16
L = 16

P = 25600
N_PAD = 4 * P
B = 128
EB_TILE = 16
E_PAD = ((E + NS * B - 1) // (NS * B)) * (NS * B)
TE = E_PAD // NS
NB = TE // B
ROWS_TILE = P // NS
ZROWS = 160


def _hop_body(ego_hbm, row_hbm, col_hbm, val_hbm, out_hbm,
              rowbuf, colbuf, valbuf, idxbuf, gbuf, zbuf, acc, sem):
    c = lax.axis_index("c")
    s = lax.axis_index("s")

    zeros = jnp.zeros((L,), jnp.float32)
    for i in range(ZROWS):
        for j in range(D // L):
            zbuf[i, pl.ds(j * L, L)] = zeros

    for p in range(2):
        base = (c * 2 + p) * P

        for i in range(ROWS_TILE // ZROWS):
            pltpu.sync_copy(zbuf, acc.at[pl.ds(s * ROWS_TILE + i * ZROWS, ZROWS)])
        plsc.subcore_barrier()

        def batch_body(i, carry):
            eb = s * TE + i * B
            pltpu.sync_copy(row_hbm.at[pl.ds(eb, B)], rowbuf)
            pltpu.sync_copy(col_hbm.at[pl.ds(eb, B)], colbuf)
            pltpu.sync_copy(val_hbm.at[pl.ds(eb, B)], valbuf)
            for g in range(B // L):
                r = rowbuf[pl.ds(g * L, L)]
                lr = r - base
                ok = (lr >= 0) & (lr < P)
                idxbuf[pl.ds(g * L, L)] = jnp.where(ok, lr, P)
            pltpu.async_copy(ego_hbm.at[colbuf], gbuf, sem).wait()
            for g in range(B // L):
                vals_g = valbuf[pl.ds(g * L, L)]
                for jj in range(L):
                    j = g * L + jj
                    vj = vals_g.at[jnp.full((L,), jj, jnp.int32)].get(
                        mode="promise_in_bounds")
                    for f in range(D // L):
                        gbuf[j, pl.ds(f * L, L)] = gbuf[j, pl.ds(f * L, L)] * vj
            pltpu.sync_copy(gbuf, acc.at[idxbuf], add=True)
            return carry

        lax.fori_loop(0, NB, batch_body, 0)
        plsc.subcore_barrier()
        pltpu.sync_copy(acc.at[pl.ds(s * ROWS_TILE, ROWS_TILE)],
                        out_hbm.at[pl.ds(base + s * ROWS_TILE, ROWS_TILE)])
        plsc.subcore_barrier()


_hop = pl.kernel(
    _hop_body,
    out_type=jax.ShapeDtypeStruct((N_PAD, D), jnp.float32),
    mesh=plsc.VectorSubcoreMesh(core_axis_name="c", subcore_axis_name="s",
                                num_cores=NC, num_subcores=NS),
    scratch_types=[
        pltpu.VMEM((B,), jnp.int32),
        pltpu.VMEM((B,), jnp.int32),
        pltpu.VMEM((B,), jnp.float32),
        pltpu.VMEM((B,), jnp.int32),
        pltpu.VMEM((B, D), jnp.float32),
        pltpu.VMEM((ZROWS, D), jnp.float32),
        pltpu.VMEM_SHARED((P + 8, D), jnp.float32),
        pltpu.SemaphoreType.DMA,
    ],
    compiler_params=pltpu.CompilerParams(use_tc_tiling_on_sc=False),
)


def kernel(user_embed, item_embed, adj_row, adj_col, adj_val):
    ego0 = jnp.concatenate([user_embed, item_embed], axis=0)
    ego0 = jnp.pad(ego0, ((0, N_PAD - N_NODES), (0, 0)))
    row = jnp.pad(adj_row.astype(jnp.int32), (0, E_PAD - E),
                  constant_values=jnp.int32(1 << 20))
    col = jnp.pad(adj_col.astype(jnp.int32), (0, E_PAD - E))
    val = jnp.pad(adj_val, (0, E_PAD - E))

    e1 = _hop(ego0, row, col, val)
    e2 = _hop(e1, row, col, val)
    e3 = _hop(e2, row, col, val)

    mean = (e1 + e2 + e3) * jnp.float32(1.0 / 3.0)
    user_all = mean[:N_USERS]
    item_all = mean[N_USERS:N_NODES]
    user_layer = e1[:N_USERS]
    item_layer = e1[N_USERS:N_NODES]
    return (user_all, item_all, user_layer, item_layer)
